# baseline (speedup 1.0000x reference)
import functools

import jax
import jax.numpy as jnp
from jax import lax
from jax.experimental import pallas as pl
from jax.experimental.pallas import tpu as pltpu
from jax.experimental.pallas import tpu_sc as plsc

E = 8
TOP_K = 2

NC = 2
NS = 16
NW = NC * NS

BM = 256
BF = 2048
BR = 512
NEG = -1e30


def _router_body(x_ref, wr_ref, br_ref,
                 eid0_ref, eid1_ref, rk0_ref, rk1_ref, c0_ref, c1_ref,
                 counts_ref, aux_ref, esum_ref, carry_ref):
    m = pl.program_id(0)
    nsteps = pl.num_programs(0)
    x = x_ref[...]
    logits = jnp.dot(x, wr_ref[...], preferred_element_type=jnp.float32)
    logits = logits + br_ref[...]

    mx = jnp.max(logits, axis=1, keepdims=True)
    ex = jnp.exp(logits - mx)
    w = ex / jnp.sum(ex, axis=1, keepdims=True)

    cols = lax.broadcasted_iota(jnp.int32, logits.shape, 1)
    m1 = jnp.max(logits, axis=1, keepdims=True)
    i1 = jnp.min(jnp.where(logits == m1, cols, E), axis=1, keepdims=True)
    sel1 = cols == i1
    l2 = jnp.where(sel1, NEG, logits)
    m2 = jnp.max(l2, axis=1, keepdims=True)
    i2 = jnp.min(jnp.where(l2 == m2, cols, E), axis=1, keepdims=True)
    sel2 = cols == i2
    sel = sel1 | sel2
    self_f = sel.astype(jnp.float32)

    v1 = jnp.sum(jnp.where(sel1, w, 0.0), axis=1, keepdims=True)
    v2 = jnp.sum(jnp.where(sel2, w, 0.0), axis=1, keepdims=True)
    norm = v1 + v2

    rows = lax.broadcasted_iota(jnp.int32, (BR, BR), 0)
    colsb = lax.broadcasted_iota(jnp.int32, (BR, BR), 1)
    tri = (rows > colsb).astype(jnp.float32)
    rank_in_blk = jnp.dot(tri, self_f, preferred_element_type=jnp.float32)

    @pl.when(m == 0)
    def _():
        carry_ref[...] = jnp.zeros_like(carry_ref)
        esum_ref[...] = jnp.zeros_like(esum_ref)

    rank = carry_ref[...] + rank_in_blk
    carry_ref[...] += jnp.sum(self_f, axis=0, keepdims=True)
    esum_ref[...] += jnp.sum(w, axis=0, keepdims=True)

    eid0_ref[...] = i1
    eid1_ref[...] = i2
    rk0_ref[...] = jnp.sum(jnp.where(sel1, rank, 0.0), axis=1,
                           keepdims=True).astype(jnp.int32)
    rk1_ref[...] = jnp.sum(jnp.where(sel2, rank, 0.0), axis=1,
                           keepdims=True).astype(jnp.int32)
    c0_ref[...] = jnp.broadcast_to(v1 / norm, (BR, 128))
    c1_ref[...] = jnp.broadcast_to(v2 / norm, (BR, 128))

    @pl.when(m == nsteps - 1)
    def _():
        counts_ref[...] = carry_ref[...].astype(jnp.int32)
        n_tok = nsteps * BR
        imp = esum_ref[...] / float(n_tok)
        dev = imp - (1.0 / E)
        aux_ref[...] = jnp.sum(dev * dev, axis=1, keepdims=True) / float(E)


def _router(Xf, Wr, br):
    N, D = Xf.shape
    grid = (N // BR,)
    col1 = lambda m: (m, 0)
    fix = lambda m: (0, 0)
    outs = pl.pallas_call(
        _router_body,
        grid=grid,
        in_specs=[
            pl.BlockSpec((BR, D), col1),
            pl.BlockSpec((D, E), fix),
            pl.BlockSpec((1, E), fix),
        ],
        out_specs=[
            pl.BlockSpec((BR, 1), col1),
            pl.BlockSpec((BR, 1), col1),
            pl.BlockSpec((BR, 1), col1),
            pl.BlockSpec((BR, 1), col1),
            pl.BlockSpec((BR, 128), col1),
            pl.BlockSpec((BR, 128), col1),
            pl.BlockSpec((1, E), fix),
            pl.BlockSpec((1, 1), fix),
        ],
        out_shape=[
            jax.ShapeDtypeStruct((N, 1), jnp.int32),
            jax.ShapeDtypeStruct((N, 1), jnp.int32),
            jax.ShapeDtypeStruct((N, 1), jnp.int32),
            jax.ShapeDtypeStruct((N, 1), jnp.int32),
            jax.ShapeDtypeStruct((N, 128), jnp.float32),
            jax.ShapeDtypeStruct((N, 128), jnp.float32),
            jax.ShapeDtypeStruct((1, E), jnp.int32),
            jax.ShapeDtypeStruct((1, 1), jnp.float32),
        ],
        scratch_shapes=[
            pltpu.VMEM((1, E), jnp.float32),
            pltpu.VMEM((1, E), jnp.float32),
        ],
    )(Xf, Wr, br.reshape(1, E))
    return outs


def _sc_scatter_body(n_chunks, cg, x_hbm, p0_hbm, p1_hbm, c0_hbm, c1_hbm,
                     out_hbm, coef_hbm, p0_v, p1_v, c0c_v, c1c_v, rows_v, sem):
    wid = lax.axis_index("s") * NC + lax.axis_index("c")
    pltpu.sync_copy(p0_hbm.at[wid], p0_v)
    pltpu.sync_copy(p1_hbm.at[wid], p1_v)
    base = wid * (n_chunks * cg)
    cbase = wid * n_chunks

    def chunk(c, _):
        pltpu.sync_copy(x_hbm.at[pl.ds(base + c * cg, cg)], rows_v)
        pltpu.sync_copy(c0_hbm.at[cbase + c], c0c_v)
        pltpu.sync_copy(c1_hbm.at[cbase + c], c1c_v)
        cp0 = pltpu.async_copy(rows_v, out_hbm.at[p0_v.at[c]], sem)
        cp1 = pltpu.async_copy(rows_v, out_hbm.at[p1_v.at[c]], sem)
        cc0 = pltpu.async_copy(c0c_v, coef_hbm.at[p0_v.at[c]], sem)
        cc1 = pltpu.async_copy(c1c_v, coef_hbm.at[p1_v.at[c]], sem)
        cp0.wait()
        cp1.wait()
        cc0.wait()
        cc1.wait()
        return 0

    lax.fori_loop(0, n_chunks, chunk, 0)


def _sc_scatter(Xf, pos0, pos1, c0x, c1x, P_cap):
    N, D = Xf.shape
    per_w = N // NW
    cg = 32
    n_chunks = per_w // cg
    p0 = pos0.reshape(NW, n_chunks, cg)
    p1 = pos1.reshape(NW, n_chunks, cg)
    c0r = c0x.reshape(NW * n_chunks, cg, 128)
    c1r = c1x.reshape(NW * n_chunks, cg, 128)
    mesh = plsc.VectorSubcoreMesh(core_axis_name="c", subcore_axis_name="s")
    return pl.kernel(
        functools.partial(_sc_scatter_body, n_chunks, cg),
        out_type=[jax.ShapeDtypeStruct((P_cap, D), jnp.float32),
                  jax.ShapeDtypeStruct((P_cap, 128), jnp.float32)],
        mesh=mesh,
        scratch_types=[
            pltpu.VMEM((n_chunks, cg), jnp.int32),
            pltpu.VMEM((n_chunks, cg), jnp.int32),
            pltpu.VMEM((cg, 128), jnp.float32),
            pltpu.VMEM((cg, 128), jnp.float32),
            pltpu.VMEM((cg, D), jnp.float32),
            pltpu.SemaphoreType.DMA,
        ],
    )(Xf, p0, p1, c0r, c1r)


def _ffn_body(be_ref, xs_ref, coef_ref, w1_ref, b1_ref, w2_ref, b2_ref,
              out_ref):
    f = pl.program_id(1)
    nf = pl.num_programs(1)
    x = xs_ref[...]
    h = jnp.dot(x, w1_ref[0], preferred_element_type=jnp.float32)
    h = jnp.maximum(h + b1_ref[0], 0.0)
    part = jnp.dot(h, w2_ref[0], preferred_element_type=jnp.float32)

    @pl.when(f == 0)
    def _():
        out_ref[...] = part

    @pl.when((f != 0) & (f != nf - 1))
    def _():
        out_ref[...] += part

    @pl.when((f == nf - 1) & (f != 0))
    def _():
        coef = coef_ref[:, :1]
        out_ref[...] = (out_ref[...] + part + b2_ref[0]) * coef


def _ffn(Xs, Coef, W1, b1, W2, b2, block_expert):
    P_cap, D = Xs.shape
    F = W1.shape[2]
    M = P_cap // BM
    NF = F // BF
    grid_spec = pltpu.PrefetchScalarGridSpec(
        num_scalar_prefetch=1,
        grid=(M, NF),
        in_specs=[
            pl.BlockSpec((BM, D), lambda m, f, be: (m, 0)),
            pl.BlockSpec((BM, 128), lambda m, f, be: (m, 0)),
            pl.BlockSpec((1, D, BF), lambda m, f, be: (be[m], 0, f)),
            pl.BlockSpec((1, 1, BF), lambda m, f, be: (be[m] * (F // BF) + f, 0, 0)),
            pl.BlockSpec((1, BF, D), lambda m, f, be: (be[m], f, 0)),
            pl.BlockSpec((1, 1, D), lambda m, f, be: (be[m], 0, 0)),
        ],
        out_specs=pl.BlockSpec((BM, D), lambda m, f, be: (m, 0)),
    )
    NFs = F // BF
    return pl.pallas_call(
        _ffn_body,
        grid_spec=grid_spec,
        out_shape=jax.ShapeDtypeStruct((P_cap, D), jnp.float32),
        compiler_params=pltpu.CompilerParams(
            dimension_semantics=("arbitrary", "arbitrary")),
    )(block_expert, Xs, Coef, W1.reshape(E, D, F), b1.reshape(E * NFs, 1, BF),
      W2.reshape(E, F, D), b2.reshape(E, 1, D))


def _sc_combine_body(n_chunks, cc, d, ys_hbm, p0_hbm, p1_hbm,
                     out_hbm, p0_v, p1_v, r0_v, r1_v, sem):
    wid = lax.axis_index("s") * NC + lax.axis_index("c")
    pltpu.sync_copy(p0_hbm.at[wid], p0_v)
    pltpu.sync_copy(p1_hbm.at[wid], p1_v)
    base = wid * (n_chunks * cc)
    nv = d // 16

    def chunk(c, _):
        cp0 = pltpu.async_copy(ys_hbm.at[p0_v.at[c]], r0_v, sem)
        cp1 = pltpu.async_copy(ys_hbm.at[p1_v.at[c]], r1_v, sem)
        cp0.wait()
        cp1.wait()

        def row(r, _):
            for j in range(nv):
                a = r0_v[r, pl.ds(j * 16, 16)]
                b = r1_v[r, pl.ds(j * 16, 16)]
                r0_v[r, pl.ds(j * 16, 16)] = a + b
            return 0

        lax.fori_loop(0, cc, row, 0)
        pltpu.sync_copy(r0_v, out_hbm.at[pl.ds(base + c * cc, cc)])
        return 0

    lax.fori_loop(0, n_chunks, chunk, 0)


def _sc_combine(Ys, pos0, pos1):
    P_cap, D = Ys.shape
    N = pos0.shape[0]
    per_w = N // NW
    cc = 16
    n_chunks = per_w // cc
    p0 = pos0.reshape(NW, n_chunks, cc)
    p1 = pos1.reshape(NW, n_chunks, cc)
    mesh = plsc.VectorSubcoreMesh(core_axis_name="c", subcore_axis_name="s")
    return pl.kernel(
        functools.partial(_sc_combine_body, n_chunks, cc, D),
        out_type=jax.ShapeDtypeStruct((N, D), jnp.float32),
        mesh=mesh,
        scratch_types=[
            pltpu.VMEM((n_chunks, cc), jnp.int32),
            pltpu.VMEM((n_chunks, cc), jnp.int32),
            pltpu.VMEM((cc, D), jnp.float32),
            pltpu.VMEM((cc, D), jnp.float32),
            pltpu.SemaphoreType.DMA,
        ],
    )(Ys, p0, p1)


def kernel(X, Wr, br, W1, b1, W2, b2):
    B, T, D = X.shape
    N = B * T
    P_cap = N * TOP_K + E * BM
    M = P_cap // BM
    Xf = X.reshape(N, D)

    eid0, eid1, rk0, rk1, c0, c1, counts, aux = _router(Xf, Wr, br)

    counts = counts.reshape(E)
    nb = (counts + BM - 1) // BM
    blk_starts = jnp.concatenate(
        [jnp.zeros((1,), jnp.int32), jnp.cumsum(nb)[:-1]]).astype(jnp.int32)
    starts = blk_starts * BM
    pos0 = (starts[eid0.reshape(-1)] + rk0.reshape(-1)).astype(jnp.int32)
    pos1 = (starts[eid1.reshape(-1)] + rk1.reshape(-1)).astype(jnp.int32)
    block_expert = (jnp.searchsorted(
        blk_starts, jnp.arange(M, dtype=jnp.int32), side="right") - 1
    ).astype(jnp.int32)
    Xs, Coef = _sc_scatter(Xf, pos0, pos1, c0, c1, P_cap)
    Ys = _ffn(Xs, Coef, W1, b1, W2, b2, block_expert)
    out = _sc_combine(Ys, pos0, pos1)
    return out.reshape(B, T, D), aux[0, 0]


# speedup vs baseline: 1.4148x; 1.4148x over previous
import functools

import jax
import jax.numpy as jnp
from jax import lax
from jax.experimental import pallas as pl
from jax.experimental.pallas import tpu as pltpu
from jax.experimental.pallas import tpu_sc as plsc

E = 8
TOP_K = 2

NC = 2
NS = 16
NW = NC * NS

BM = 512
BF = 2048
BR = 512
NEG = -1e30


def _router_body(x_ref, wr_ref, br_ref,
                 eid0_ref, eid1_ref, rk0_ref, rk1_ref, c0_ref, c1_ref,
                 counts_ref, aux_ref, esum_ref, carry_ref):
    m = pl.program_id(0)
    nsteps = pl.num_programs(0)
    x = x_ref[...]
    logits = jnp.dot(x, wr_ref[...], preferred_element_type=jnp.float32)
    logits = logits + br_ref[...]

    mx = jnp.max(logits, axis=1, keepdims=True)
    ex = jnp.exp(logits - mx)
    w = ex / jnp.sum(ex, axis=1, keepdims=True)

    cols = lax.broadcasted_iota(jnp.int32, logits.shape, 1)
    m1 = jnp.max(logits, axis=1, keepdims=True)
    i1 = jnp.min(jnp.where(logits == m1, cols, E), axis=1, keepdims=True)
    sel1 = cols == i1
    l2 = jnp.where(sel1, NEG, logits)
    m2 = jnp.max(l2, axis=1, keepdims=True)
    i2 = jnp.min(jnp.where(l2 == m2, cols, E), axis=1, keepdims=True)
    sel2 = cols == i2
    sel = sel1 | sel2
    self_f = sel.astype(jnp.float32)

    v1 = jnp.sum(jnp.where(sel1, w, 0.0), axis=1, keepdims=True)
    v2 = jnp.sum(jnp.where(sel2, w, 0.0), axis=1, keepdims=True)
    norm = v1 + v2

    rows = lax.broadcasted_iota(jnp.int32, (BR, BR), 0)
    colsb = lax.broadcasted_iota(jnp.int32, (BR, BR), 1)
    tri = (rows > colsb).astype(jnp.float32)
    rank_in_blk = jnp.dot(tri, self_f, preferred_element_type=jnp.float32)

    @pl.when(m == 0)
    def _():
        carry_ref[...] = jnp.zeros_like(carry_ref)
        esum_ref[...] = jnp.zeros_like(esum_ref)

    rank = carry_ref[...] + rank_in_blk
    carry_ref[...] += jnp.sum(self_f, axis=0, keepdims=True)
    esum_ref[...] += jnp.sum(w, axis=0, keepdims=True)

    eid0_ref[...] = i1
    eid1_ref[...] = i2
    rk0_ref[...] = jnp.sum(jnp.where(sel1, rank, 0.0), axis=1,
                           keepdims=True).astype(jnp.int32)
    rk1_ref[...] = jnp.sum(jnp.where(sel2, rank, 0.0), axis=1,
                           keepdims=True).astype(jnp.int32)
    c0_ref[...] = jnp.broadcast_to(v1 / norm, (BR, 128))
    c1_ref[...] = jnp.broadcast_to(v2 / norm, (BR, 128))

    @pl.when(m == nsteps - 1)
    def _():
        counts_ref[...] = carry_ref[...].astype(jnp.int32)
        n_tok = nsteps * BR
        imp = esum_ref[...] / float(n_tok)
        dev = imp - (1.0 / E)
        aux_ref[...] = jnp.sum(dev * dev, axis=1, keepdims=True) / float(E)


def _router(Xf, Wr, br):
    N, D = Xf.shape
    grid = (N // BR,)
    col1 = lambda m: (m, 0)
    fix = lambda m: (0, 0)
    outs = pl.pallas_call(
        _router_body,
        grid=grid,
        in_specs=[
            pl.BlockSpec((BR, D), col1),
            pl.BlockSpec((D, E), fix),
            pl.BlockSpec((1, E), fix),
        ],
        out_specs=[
            pl.BlockSpec((BR, 1), col1),
            pl.BlockSpec((BR, 1), col1),
            pl.BlockSpec((BR, 1), col1),
            pl.BlockSpec((BR, 1), col1),
            pl.BlockSpec((BR, 128), col1),
            pl.BlockSpec((BR, 128), col1),
            pl.BlockSpec((1, E), fix),
            pl.BlockSpec((1, 1), fix),
        ],
        out_shape=[
            jax.ShapeDtypeStruct((N, 1), jnp.int32),
            jax.ShapeDtypeStruct((N, 1), jnp.int32),
            jax.ShapeDtypeStruct((N, 1), jnp.int32),
            jax.ShapeDtypeStruct((N, 1), jnp.int32),
            jax.ShapeDtypeStruct((N, 128), jnp.float32),
            jax.ShapeDtypeStruct((N, 128), jnp.float32),
            jax.ShapeDtypeStruct((1, E), jnp.int32),
            jax.ShapeDtypeStruct((1, 1), jnp.float32),
        ],
        scratch_shapes=[
            pltpu.VMEM((1, E), jnp.float32),
            pltpu.VMEM((1, E), jnp.float32),
        ],
    )(Xf, Wr, br.reshape(1, E))
    return outs


def _sc_scatter_body(n_chunks, cg, x_hbm, p0_hbm, p1_hbm, c0_hbm, c1_hbm,
                     out_hbm, coef_hbm, p0_v, p1_v, c0c_v, c1c_v, rows_v, sem):
    wid = lax.axis_index("s") * NC + lax.axis_index("c")
    pltpu.sync_copy(p0_hbm.at[wid], p0_v)
    pltpu.sync_copy(p1_hbm.at[wid], p1_v)
    base = wid * (n_chunks * cg)
    cbase = wid * n_chunks

    def chunk(c, _):
        pltpu.sync_copy(x_hbm.at[pl.ds(base + c * cg, cg)], rows_v)
        pltpu.sync_copy(c0_hbm.at[cbase + c], c0c_v)
        pltpu.sync_copy(c1_hbm.at[cbase + c], c1c_v)
        cp0 = pltpu.async_copy(rows_v, out_hbm.at[p0_v.at[c]], sem)
        cp1 = pltpu.async_copy(rows_v, out_hbm.at[p1_v.at[c]], sem)
        cc0 = pltpu.async_copy(c0c_v, coef_hbm.at[p0_v.at[c]], sem)
        cc1 = pltpu.async_copy(c1c_v, coef_hbm.at[p1_v.at[c]], sem)
        cp0.wait()
        cp1.wait()
        cc0.wait()
        cc1.wait()
        return 0

    lax.fori_loop(0, n_chunks, chunk, 0)


def _sc_scatter(Xf, pos0, pos1, c0x, c1x, P_cap):
    N, D = Xf.shape
    per_w = N // NW
    cg = 32
    n_chunks = per_w // cg
    p0 = pos0.reshape(NW, n_chunks, cg)
    p1 = pos1.reshape(NW, n_chunks, cg)
    c0r = c0x.reshape(NW * n_chunks, cg, 128)
    c1r = c1x.reshape(NW * n_chunks, cg, 128)
    mesh = plsc.VectorSubcoreMesh(core_axis_name="c", subcore_axis_name="s")
    return pl.kernel(
        functools.partial(_sc_scatter_body, n_chunks, cg),
        out_type=[jax.ShapeDtypeStruct((P_cap, D), jnp.float32),
                  jax.ShapeDtypeStruct((P_cap, 128), jnp.float32)],
        mesh=mesh,
        scratch_types=[
            pltpu.VMEM((n_chunks, cg), jnp.int32),
            pltpu.VMEM((n_chunks, cg), jnp.int32),
            pltpu.VMEM((cg, 128), jnp.float32),
            pltpu.VMEM((cg, 128), jnp.float32),
            pltpu.VMEM((cg, D), jnp.float32),
            pltpu.SemaphoreType.DMA,
        ],
    )(Xf, p0, p1, c0r, c1r)


def _ffn_body(be_ref, xs_ref, coef_ref, w1_ref, b1_ref, w2_ref, b2_ref,
              out_ref):
    f = pl.program_id(1)
    nf = pl.num_programs(1)
    x = xs_ref[...]
    h = jnp.dot(x, w1_ref[0], preferred_element_type=jnp.float32)
    h = jnp.maximum(h + b1_ref[0], 0.0)
    part = jnp.dot(h, w2_ref[0], preferred_element_type=jnp.float32)

    @pl.when(f == 0)
    def _():
        out_ref[...] = part

    @pl.when((f != 0) & (f != nf - 1))
    def _():
        out_ref[...] += part

    @pl.when((f == nf - 1) & (f != 0))
    def _():
        coef = coef_ref[:, :1]
        out_ref[...] = (out_ref[...] + part + b2_ref[0]) * coef


def _ffn(Xs, Coef, W1, b1, W2, b2, block_expert):
    P_cap, D = Xs.shape
    F = W1.shape[2]
    M = P_cap // BM
    NF = F // BF
    grid_spec = pltpu.PrefetchScalarGridSpec(
        num_scalar_prefetch=1,
        grid=(M, NF),
        in_specs=[
            pl.BlockSpec((BM, D), lambda m, f, be: (m, 0)),
            pl.BlockSpec((BM, 128), lambda m, f, be: (m, 0)),
            pl.BlockSpec((1, D, BF), lambda m, f, be: (be[m], 0, f)),
            pl.BlockSpec((1, 1, BF), lambda m, f, be: (be[m] * (F // BF) + f, 0, 0)),
            pl.BlockSpec((1, BF, D), lambda m, f, be: (be[m], f, 0)),
            pl.BlockSpec((1, 1, D), lambda m, f, be: (be[m], 0, 0)),
        ],
        out_specs=pl.BlockSpec((BM, D), lambda m, f, be: (m, 0)),
    )
    NFs = F // BF
    return pl.pallas_call(
        _ffn_body,
        grid_spec=grid_spec,
        out_shape=jax.ShapeDtypeStruct((P_cap, D), jnp.float32),
        compiler_params=pltpu.CompilerParams(
            dimension_semantics=("arbitrary", "arbitrary")),
    )(block_expert, Xs, Coef, W1.reshape(E, D, F), b1.reshape(E * NFs, 1, BF),
      W2.reshape(E, F, D), b2.reshape(E, 1, D))


def _sc_combine_body(n_chunks, cc, d, ys_hbm, p0_hbm, p1_hbm,
                     out_hbm, p0_v, p1_v, r0_v, r1_v, sem):
    wid = lax.axis_index("s") * NC + lax.axis_index("c")
    pltpu.sync_copy(p0_hbm.at[wid], p0_v)
    pltpu.sync_copy(p1_hbm.at[wid], p1_v)
    base = wid * (n_chunks * cc)
    nv = d // 16

    def chunk(c, _):
        cp0 = pltpu.async_copy(ys_hbm.at[p0_v.at[c]], r0_v, sem)
        cp1 = pltpu.async_copy(ys_hbm.at[p1_v.at[c]], r1_v, sem)
        cp0.wait()
        cp1.wait()

        def row(r, _):
            for j in range(nv):
                a = r0_v[r, pl.ds(j * 16, 16)]
                b = r1_v[r, pl.ds(j * 16, 16)]
                r0_v[r, pl.ds(j * 16, 16)] = a + b
            return 0

        lax.fori_loop(0, cc, row, 0)
        pltpu.sync_copy(r0_v, out_hbm.at[pl.ds(base + c * cc, cc)])
        return 0

    lax.fori_loop(0, n_chunks, chunk, 0)


def _sc_combine(Ys, pos0, pos1):
    P_cap, D = Ys.shape
    N = pos0.shape[0]
    per_w = N // NW
    cc = 16
    n_chunks = per_w // cc
    p0 = pos0.reshape(NW, n_chunks, cc)
    p1 = pos1.reshape(NW, n_chunks, cc)
    mesh = plsc.VectorSubcoreMesh(core_axis_name="c", subcore_axis_name="s")
    return pl.kernel(
        functools.partial(_sc_combine_body, n_chunks, cc, D),
        out_type=jax.ShapeDtypeStruct((N, D), jnp.float32),
        mesh=mesh,
        scratch_types=[
            pltpu.VMEM((n_chunks, cc), jnp.int32),
            pltpu.VMEM((n_chunks, cc), jnp.int32),
            pltpu.VMEM((cc, D), jnp.float32),
            pltpu.VMEM((cc, D), jnp.float32),
            pltpu.SemaphoreType.DMA,
        ],
    )(Ys, p0, p1)


def kernel(X, Wr, br, W1, b1, W2, b2):
    B, T, D = X.shape
    N = B * T
    P_cap = N * TOP_K + E * BM
    M = P_cap // BM
    Xf = X.reshape(N, D)

    eid0, eid1, rk0, rk1, c0, c1, counts, aux = _router(Xf, Wr, br)

    counts = counts.reshape(E)
    nb = (counts + BM - 1) // BM
    blk_starts = jnp.concatenate(
        [jnp.zeros((1,), jnp.int32), jnp.cumsum(nb)[:-1]]).astype(jnp.int32)
    starts = blk_starts * BM
    pos0 = (starts[eid0.reshape(-1)] + rk0.reshape(-1)).astype(jnp.int32)
    pos1 = (starts[eid1.reshape(-1)] + rk1.reshape(-1)).astype(jnp.int32)
    block_expert = (jnp.searchsorted(
        blk_starts, jnp.arange(M, dtype=jnp.int32), side="right") - 1
    ).astype(jnp.int32)
    Xs, Coef = _sc_scatter(Xf, pos0, pos1, c0, c1, P_cap)
    Ys = _ffn(Xs, Coef, W1, b1, W2, b2, block_expert)
    out = _sc_combine(Ys, pos0, pos1)
    return out.reshape(B, T, D), aux[0, 0]


# speedup vs baseline: 1.4673x; 1.0372x over previous
import functools

import jax
import jax.numpy as jnp
from jax import lax
from jax.experimental import pallas as pl
from jax.experimental.pallas import tpu as pltpu
from jax.experimental.pallas import tpu_sc as plsc

E = 8
TOP_K = 2

NC = 2
NS = 16
NW = NC * NS

BM = 512
BF = 2048
BR = 512
NEG = -1e30


def _router_body(x_ref, wr_ref, br_ref,
                 eid0_ref, eid1_ref, rk0_ref, rk1_ref, c0_ref, c1_ref,
                 counts_ref, aux_ref, esum_ref, carry_ref):
    m = pl.program_id(0)
    nsteps = pl.num_programs(0)
    x = x_ref[...]
    logits = jnp.dot(x, wr_ref[...], preferred_element_type=jnp.float32)
    logits = logits + br_ref[...]

    mx = jnp.max(logits, axis=1, keepdims=True)
    ex = jnp.exp(logits - mx)
    w = ex / jnp.sum(ex, axis=1, keepdims=True)

    cols = lax.broadcasted_iota(jnp.int32, logits.shape, 1)
    m1 = jnp.max(logits, axis=1, keepdims=True)
    i1 = jnp.min(jnp.where(logits == m1, cols, E), axis=1, keepdims=True)
    sel1 = cols == i1
    l2 = jnp.where(sel1, NEG, logits)
    m2 = jnp.max(l2, axis=1, keepdims=True)
    i2 = jnp.min(jnp.where(l2 == m2, cols, E), axis=1, keepdims=True)
    sel2 = cols == i2
    sel = sel1 | sel2
    self_f = sel.astype(jnp.float32)

    v1 = jnp.sum(jnp.where(sel1, w, 0.0), axis=1, keepdims=True)
    v2 = jnp.sum(jnp.where(sel2, w, 0.0), axis=1, keepdims=True)
    norm = v1 + v2

    rows = lax.broadcasted_iota(jnp.int32, (BR, BR), 0)
    colsb = lax.broadcasted_iota(jnp.int32, (BR, BR), 1)
    tri = (rows > colsb).astype(jnp.float32)
    rank_in_blk = jnp.dot(tri, self_f, preferred_element_type=jnp.float32)

    @pl.when(m == 0)
    def _():
        carry_ref[...] = jnp.zeros_like(carry_ref)
        esum_ref[...] = jnp.zeros_like(esum_ref)

    rank = carry_ref[...] + rank_in_blk
    carry_ref[...] += jnp.sum(self_f, axis=0, keepdims=True)
    esum_ref[...] += jnp.sum(w, axis=0, keepdims=True)

    eid0_ref[...] = i1
    eid1_ref[...] = i2
    rk0_ref[...] = jnp.sum(jnp.where(sel1, rank, 0.0), axis=1,
                           keepdims=True).astype(jnp.int32)
    rk1_ref[...] = jnp.sum(jnp.where(sel2, rank, 0.0), axis=1,
                           keepdims=True).astype(jnp.int32)
    c0_ref[...] = jnp.broadcast_to(v1 / norm, (BR, 128))
    c1_ref[...] = jnp.broadcast_to(v2 / norm, (BR, 128))

    @pl.when(m == nsteps - 1)
    def _():
        counts_ref[...] = carry_ref[...].astype(jnp.int32)
        n_tok = nsteps * BR
        imp = esum_ref[...] / float(n_tok)
        dev = imp - (1.0 / E)
        aux_ref[...] = jnp.sum(dev * dev, axis=1, keepdims=True) / float(E)


def _router(Xf, Wr, br):
    N, D = Xf.shape
    grid = (N // BR,)
    col1 = lambda m: (m, 0)
    fix = lambda m: (0, 0)
    outs = pl.pallas_call(
        _router_body,
        grid=grid,
        in_specs=[
            pl.BlockSpec((BR, D), col1),
            pl.BlockSpec((D, E), fix),
            pl.BlockSpec((1, E), fix),
        ],
        out_specs=[
            pl.BlockSpec((BR, 1), col1),
            pl.BlockSpec((BR, 1), col1),
            pl.BlockSpec((BR, 1), col1),
            pl.BlockSpec((BR, 1), col1),
            pl.BlockSpec((BR, 128), col1),
            pl.BlockSpec((BR, 128), col1),
            pl.BlockSpec((1, E), fix),
            pl.BlockSpec((1, 1), fix),
        ],
        out_shape=[
            jax.ShapeDtypeStruct((N, 1), jnp.int32),
            jax.ShapeDtypeStruct((N, 1), jnp.int32),
            jax.ShapeDtypeStruct((N, 1), jnp.int32),
            jax.ShapeDtypeStruct((N, 1), jnp.int32),
            jax.ShapeDtypeStruct((N, 128), jnp.float32),
            jax.ShapeDtypeStruct((N, 128), jnp.float32),
            jax.ShapeDtypeStruct((1, E), jnp.int32),
            jax.ShapeDtypeStruct((1, 1), jnp.float32),
        ],
        scratch_shapes=[
            pltpu.VMEM((1, E), jnp.float32),
            pltpu.VMEM((1, E), jnp.float32),
        ],
    )(Xf, Wr, br.reshape(1, E))
    return outs


def _sc_scatter_body(n_chunks, cg, x_hbm, p0_hbm, p1_hbm, c0_hbm, c1_hbm,
                     out_hbm, coef_hbm, p0_v, p1_v, c0c_v, c1c_v, rows_v, sem):
    wid = lax.axis_index("s") * NC + lax.axis_index("c")
    pltpu.sync_copy(p0_hbm.at[wid], p0_v)
    pltpu.sync_copy(p1_hbm.at[wid], p1_v)
    base = wid * (n_chunks * cg)
    cbase = wid * n_chunks

    def chunk(c, _):
        pltpu.sync_copy(x_hbm.at[pl.ds(base + c * cg, cg)], rows_v)
        pltpu.sync_copy(c0_hbm.at[cbase + c], c0c_v)
        pltpu.sync_copy(c1_hbm.at[cbase + c], c1c_v)
        cp0 = pltpu.async_copy(rows_v, out_hbm.at[p0_v.at[c]], sem)
        cp1 = pltpu.async_copy(rows_v, out_hbm.at[p1_v.at[c]], sem)
        cc0 = pltpu.async_copy(c0c_v, coef_hbm.at[p0_v.at[c]], sem)
        cc1 = pltpu.async_copy(c1c_v, coef_hbm.at[p1_v.at[c]], sem)
        cp0.wait()
        cp1.wait()
        cc0.wait()
        cc1.wait()
        return 0

    lax.fori_loop(0, n_chunks, chunk, 0)


def _sc_scatter(Xf, pos0, pos1, c0x, c1x, P_cap):
    N, D = Xf.shape
    per_w = N // NW
    cg = 64
    n_chunks = per_w // cg
    p0 = pos0.reshape(NW, n_chunks, cg)
    p1 = pos1.reshape(NW, n_chunks, cg)
    c0r = c0x.reshape(NW * n_chunks, cg, 128)
    c1r = c1x.reshape(NW * n_chunks, cg, 128)
    mesh = plsc.VectorSubcoreMesh(core_axis_name="c", subcore_axis_name="s")
    return pl.kernel(
        functools.partial(_sc_scatter_body, n_chunks, cg),
        out_type=[jax.ShapeDtypeStruct((P_cap, D), jnp.float32),
                  jax.ShapeDtypeStruct((P_cap, 128), jnp.float32)],
        mesh=mesh,
        scratch_types=[
            pltpu.VMEM((n_chunks, cg), jnp.int32),
            pltpu.VMEM((n_chunks, cg), jnp.int32),
            pltpu.VMEM((cg, 128), jnp.float32),
            pltpu.VMEM((cg, 128), jnp.float32),
            pltpu.VMEM((cg, D), jnp.float32),
            pltpu.SemaphoreType.DMA,
        ],
    )(Xf, p0, p1, c0r, c1r)


def _ffn_body(be_ref, xs_ref, coef_ref, w1_ref, b1_ref, w2_ref, b2_ref,
              out_ref):
    m = pl.program_id(0)
    f = pl.program_id(1)
    nf = pl.num_programs(1)
    n_real = be_ref[pl.num_programs(0)]

    @pl.when(m < n_real)
    def _():
        x = xs_ref[...]
        h = jnp.dot(x, w1_ref[0], preferred_element_type=jnp.float32)
        h = jnp.maximum(h + b1_ref[0], 0.0)
        part = jnp.dot(h, w2_ref[0], preferred_element_type=jnp.float32)

        @pl.when(f == 0)
        def _():
            out_ref[...] = part

        @pl.when((f != 0) & (f != nf - 1))
        def _():
            out_ref[...] += part

        @pl.when((f == nf - 1) & (f != 0))
        def _():
            coef = coef_ref[:, :1]
            out_ref[...] = (out_ref[...] + part + b2_ref[0]) * coef


def _ffn(Xs, Coef, W1, b1, W2, b2, block_expert_ext):
    P_cap, D = Xs.shape
    F = W1.shape[2]
    M = P_cap // BM
    NF = F // BF
    grid_spec = pltpu.PrefetchScalarGridSpec(
        num_scalar_prefetch=1,
        grid=(M, NF),
        in_specs=[
            pl.BlockSpec((BM, D), lambda m, f, be: (m, 0)),
            pl.BlockSpec((BM, 128), lambda m, f, be: (m, 0)),
            pl.BlockSpec((1, D, BF), lambda m, f, be: (be[m], 0, f)),
            pl.BlockSpec((1, 1, BF), lambda m, f, be: (be[m] * (F // BF) + f, 0, 0)),
            pl.BlockSpec((1, BF, D), lambda m, f, be: (be[m], f, 0)),
            pl.BlockSpec((1, 1, D), lambda m, f, be: (be[m], 0, 0)),
        ],
        out_specs=pl.BlockSpec((BM, D), lambda m, f, be: (m, 0)),
    )
    NFs = F // BF
    return pl.pallas_call(
        _ffn_body,
        grid_spec=grid_spec,
        out_shape=jax.ShapeDtypeStruct((P_cap, D), jnp.float32),
        compiler_params=pltpu.CompilerParams(
            dimension_semantics=("arbitrary", "arbitrary")),
    )(block_expert_ext, Xs, Coef, W1.reshape(E, D, F),
      b1.reshape(E * NFs, 1, BF), W2.reshape(E, F, D), b2.reshape(E, 1, D))


def _sc_combine_body(n_chunks, cc, d, ys_hbm, p0_hbm, p1_hbm,
                     out_hbm, p0_v, p1_v, r0_v, r1_v, sem):
    wid = lax.axis_index("s") * NC + lax.axis_index("c")
    pltpu.sync_copy(p0_hbm.at[wid], p0_v)
    pltpu.sync_copy(p1_hbm.at[wid], p1_v)
    base = wid * (n_chunks * cc)
    nv = d // 16

    def chunk(c, _):
        cp0 = pltpu.async_copy(ys_hbm.at[p0_v.at[c]], r0_v, sem)
        cp1 = pltpu.async_copy(ys_hbm.at[p1_v.at[c]], r1_v, sem)
        cp0.wait()
        cp1.wait()

        def row(r, _):
            for j in range(nv):
                a = r0_v[r, pl.ds(j * 16, 16)]
                b = r1_v[r, pl.ds(j * 16, 16)]
                r0_v[r, pl.ds(j * 16, 16)] = a + b
            return 0

        lax.fori_loop(0, cc, row, 0)
        pltpu.sync_copy(r0_v, out_hbm.at[pl.ds(base + c * cc, cc)])
        return 0

    lax.fori_loop(0, n_chunks, chunk, 0)


def _sc_combine(Ys, pos0, pos1):
    P_cap, D = Ys.shape
    N = pos0.shape[0]
    per_w = N // NW
    cc = 32
    n_chunks = per_w // cc
    p0 = pos0.reshape(NW, n_chunks, cc)
    p1 = pos1.reshape(NW, n_chunks, cc)
    mesh = plsc.VectorSubcoreMesh(core_axis_name="c", subcore_axis_name="s")
    return pl.kernel(
        functools.partial(_sc_combine_body, n_chunks, cc, D),
        out_type=jax.ShapeDtypeStruct((N, D), jnp.float32),
        mesh=mesh,
        scratch_types=[
            pltpu.VMEM((n_chunks, cc), jnp.int32),
            pltpu.VMEM((n_chunks, cc), jnp.int32),
            pltpu.VMEM((cc, D), jnp.float32),
            pltpu.VMEM((cc, D), jnp.float32),
            pltpu.SemaphoreType.DMA,
        ],
    )(Ys, p0, p1)


def kernel(X, Wr, br, W1, b1, W2, b2):
    B, T, D = X.shape
    N = B * T
    P_cap = N * TOP_K + E * BM
    M = P_cap // BM
    Xf = X.reshape(N, D)

    eid0, eid1, rk0, rk1, c0, c1, counts, aux = _router(Xf, Wr, br)

    counts = counts.reshape(E)
    nb = (counts + BM - 1) // BM
    blk_starts = jnp.concatenate(
        [jnp.zeros((1,), jnp.int32), jnp.cumsum(nb)[:-1]]).astype(jnp.int32)
    starts = blk_starts * BM
    pos0 = (starts[eid0.reshape(-1)] + rk0.reshape(-1)).astype(jnp.int32)
    pos1 = (starts[eid1.reshape(-1)] + rk1.reshape(-1)).astype(jnp.int32)
    block_expert = (jnp.searchsorted(
        blk_starts, jnp.arange(M, dtype=jnp.int32), side="right") - 1
    ).astype(jnp.int32)
    n_real = jnp.sum(nb).astype(jnp.int32)
    block_expert_ext = jnp.concatenate([block_expert, n_real[None]])
    Xs, Coef = _sc_scatter(Xf, pos0, pos1, c0, c1, P_cap)
    Ys = _ffn(Xs, Coef, W1, b1, W2, b2, block_expert_ext)
    out = _sc_combine(Ys, pos0, pos1)
    return out.reshape(B, T, D), aux[0, 0]


# speedup vs baseline: 1.4698x; 1.0017x over previous
import functools

import jax
import jax.numpy as jnp
from jax import lax
from jax.experimental import pallas as pl
from jax.experimental.pallas import tpu as pltpu
from jax.experimental.pallas import tpu_sc as plsc

E = 8
TOP_K = 2

NC = 2
NS = 16
NW = NC * NS

BM = 512
BF = 2048
BR = 1024
NEG = -1e30


def _router_body(x_ref, wr_ref, br_ref,
                 eid0_ref, eid1_ref, rk0_ref, rk1_ref, c0_ref, c1_ref,
                 counts_ref, aux_ref, esum_ref, carry_ref):
    m = pl.program_id(0)
    nsteps = pl.num_programs(0)
    x = x_ref[...]
    logits = jnp.dot(x, wr_ref[...], preferred_element_type=jnp.float32)
    logits = logits + br_ref[...]

    mx = jnp.max(logits, axis=1, keepdims=True)
    ex = jnp.exp(logits - mx)
    w = ex / jnp.sum(ex, axis=1, keepdims=True)

    cols = lax.broadcasted_iota(jnp.int32, logits.shape, 1)
    m1 = jnp.max(logits, axis=1, keepdims=True)
    i1 = jnp.min(jnp.where(logits == m1, cols, E), axis=1, keepdims=True)
    sel1 = cols == i1
    l2 = jnp.where(sel1, NEG, logits)
    m2 = jnp.max(l2, axis=1, keepdims=True)
    i2 = jnp.min(jnp.where(l2 == m2, cols, E), axis=1, keepdims=True)
    sel2 = cols == i2
    sel = sel1 | sel2
    self_f = sel.astype(jnp.float32)

    v1 = jnp.sum(jnp.where(sel1, w, 0.0), axis=1, keepdims=True)
    v2 = jnp.sum(jnp.where(sel2, w, 0.0), axis=1, keepdims=True)
    norm = v1 + v2

    rows = lax.broadcasted_iota(jnp.int32, (BR, BR), 0)
    colsb = lax.broadcasted_iota(jnp.int32, (BR, BR), 1)
    tri = (rows > colsb).astype(jnp.float32)
    rank_in_blk = jnp.dot(tri, self_f, preferred_element_type=jnp.float32)

    @pl.when(m == 0)
    def _():
        carry_ref[...] = jnp.zeros_like(carry_ref)
        esum_ref[...] = jnp.zeros_like(esum_ref)

    rank = carry_ref[...] + rank_in_blk
    carry_ref[...] += jnp.sum(self_f, axis=0, keepdims=True)
    esum_ref[...] += jnp.sum(w, axis=0, keepdims=True)

    eid0_ref[...] = i1
    eid1_ref[...] = i2
    rk0_ref[...] = jnp.sum(jnp.where(sel1, rank, 0.0), axis=1,
                           keepdims=True).astype(jnp.int32)
    rk1_ref[...] = jnp.sum(jnp.where(sel2, rank, 0.0), axis=1,
                           keepdims=True).astype(jnp.int32)
    c0_ref[...] = jnp.broadcast_to(v1 / norm, (BR, 128))
    c1_ref[...] = jnp.broadcast_to(v2 / norm, (BR, 128))

    @pl.when(m == nsteps - 1)
    def _():
        counts_ref[...] = carry_ref[...].astype(jnp.int32)
        n_tok = nsteps * BR
        imp = esum_ref[...] / float(n_tok)
        dev = imp - (1.0 / E)
        aux_ref[...] = jnp.sum(dev * dev, axis=1, keepdims=True) / float(E)


def _router(Xf, Wr, br):
    N, D = Xf.shape
    grid = (N // BR,)
    col1 = lambda m: (m, 0)
    fix = lambda m: (0, 0)
    outs = pl.pallas_call(
        _router_body,
        grid=grid,
        in_specs=[
            pl.BlockSpec((BR, D), col1),
            pl.BlockSpec((D, E), fix),
            pl.BlockSpec((1, E), fix),
        ],
        out_specs=[
            pl.BlockSpec((BR, 1), col1),
            pl.BlockSpec((BR, 1), col1),
            pl.BlockSpec((BR, 1), col1),
            pl.BlockSpec((BR, 1), col1),
            pl.BlockSpec((BR, 128), col1),
            pl.BlockSpec((BR, 128), col1),
            pl.BlockSpec((1, E), fix),
            pl.BlockSpec((1, 1), fix),
        ],
        out_shape=[
            jax.ShapeDtypeStruct((N, 1), jnp.int32),
            jax.ShapeDtypeStruct((N, 1), jnp.int32),
            jax.ShapeDtypeStruct((N, 1), jnp.int32),
            jax.ShapeDtypeStruct((N, 1), jnp.int32),
            jax.ShapeDtypeStruct((N, 128), jnp.float32),
            jax.ShapeDtypeStruct((N, 128), jnp.float32),
            jax.ShapeDtypeStruct((1, E), jnp.int32),
            jax.ShapeDtypeStruct((1, 1), jnp.float32),
        ],
        scratch_shapes=[
            pltpu.VMEM((1, E), jnp.float32),
            pltpu.VMEM((1, E), jnp.float32),
        ],
    )(Xf, Wr, br.reshape(1, E))
    return outs


def _sc_scatter_body(n_chunks, cg, x_hbm, p0_hbm, p1_hbm, c0_hbm, c1_hbm,
                     out_hbm, coef_hbm, p0_v, p1_v, c0c_v, c1c_v, rows_v, sem):
    wid = lax.axis_index("s") * NC + lax.axis_index("c")
    pltpu.sync_copy(p0_hbm.at[wid], p0_v)
    pltpu.sync_copy(p1_hbm.at[wid], p1_v)
    base = wid * (n_chunks * cg)
    cbase = wid * n_chunks

    def chunk(c, _):
        pltpu.sync_copy(x_hbm.at[pl.ds(base + c * cg, cg)], rows_v)
        pltpu.sync_copy(c0_hbm.at[cbase + c], c0c_v)
        pltpu.sync_copy(c1_hbm.at[cbase + c], c1c_v)
        cp0 = pltpu.async_copy(rows_v, out_hbm.at[p0_v.at[c]], sem)
        cp1 = pltpu.async_copy(rows_v, out_hbm.at[p1_v.at[c]], sem)
        cc0 = pltpu.async_copy(c0c_v, coef_hbm.at[p0_v.at[c]], sem)
        cc1 = pltpu.async_copy(c1c_v, coef_hbm.at[p1_v.at[c]], sem)
        cp0.wait()
        cp1.wait()
        cc0.wait()
        cc1.wait()
        return 0

    lax.fori_loop(0, n_chunks, chunk, 0)


def _sc_scatter(Xf, pos0, pos1, c0x, c1x, P_cap):
    N, D = Xf.shape
    per_w = N // NW
    cg = 64
    n_chunks = per_w // cg
    p0 = pos0.reshape(NW, n_chunks, cg)
    p1 = pos1.reshape(NW, n_chunks, cg)
    c0r = c0x.reshape(NW * n_chunks, cg, 128)
    c1r = c1x.reshape(NW * n_chunks, cg, 128)
    mesh = plsc.VectorSubcoreMesh(core_axis_name="c", subcore_axis_name="s")
    return pl.kernel(
        functools.partial(_sc_scatter_body, n_chunks, cg),
        out_type=[jax.ShapeDtypeStruct((P_cap, D), jnp.float32),
                  jax.ShapeDtypeStruct((P_cap, 128), jnp.float32)],
        mesh=mesh,
        scratch_types=[
            pltpu.VMEM((n_chunks, cg), jnp.int32),
            pltpu.VMEM((n_chunks, cg), jnp.int32),
            pltpu.VMEM((cg, 128), jnp.float32),
            pltpu.VMEM((cg, 128), jnp.float32),
            pltpu.VMEM((cg, D), jnp.float32),
            pltpu.SemaphoreType.DMA,
        ],
    )(Xf, p0, p1, c0r, c1r)


def _ffn_body(be_ref, xs_ref, coef_ref, w1_ref, b1_ref, w2_ref, b2_ref,
              out_ref):
    m = pl.program_id(0)
    f = pl.program_id(1)
    nf = pl.num_programs(1)
    n_real = be_ref[pl.num_programs(0)]

    @pl.when(m < n_real)
    def _():
        x = xs_ref[...]
        h = jnp.dot(x, w1_ref[0], preferred_element_type=jnp.float32)
        h = jnp.maximum(h + b1_ref[0], 0.0)
        part = jnp.dot(h, w2_ref[0], preferred_element_type=jnp.float32)

        @pl.when(f == 0)
        def _():
            out_ref[...] = part

        @pl.when((f != 0) & (f != nf - 1))
        def _():
            out_ref[...] += part

        @pl.when((f == nf - 1) & (f != 0))
        def _():
            coef = coef_ref[:, :1]
            out_ref[...] = (out_ref[...] + part + b2_ref[0]) * coef


def _ffn(Xs, Coef, W1, b1, W2, b2, block_expert_ext):
    P_cap, D = Xs.shape
    F = W1.shape[2]
    M = P_cap // BM
    NF = F // BF
    grid_spec = pltpu.PrefetchScalarGridSpec(
        num_scalar_prefetch=1,
        grid=(M, NF),
        in_specs=[
            pl.BlockSpec((BM, D), lambda m, f, be: (m, 0)),
            pl.BlockSpec((BM, 128), lambda m, f, be: (m, 0)),
            pl.BlockSpec((1, D, BF), lambda m, f, be: (be[m], 0, f)),
            pl.BlockSpec((1, 1, BF), lambda m, f, be: (be[m] * (F // BF) + f, 0, 0)),
            pl.BlockSpec((1, BF, D), lambda m, f, be: (be[m], f, 0)),
            pl.BlockSpec((1, 1, D), lambda m, f, be: (be[m], 0, 0)),
        ],
        out_specs=pl.BlockSpec((BM, D), lambda m, f, be: (m, 0)),
    )
    NFs = F // BF
    return pl.pallas_call(
        _ffn_body,
        grid_spec=grid_spec,
        out_shape=jax.ShapeDtypeStruct((P_cap, D), jnp.float32),
        compiler_params=pltpu.CompilerParams(
            dimension_semantics=("arbitrary", "arbitrary")),
    )(block_expert_ext, Xs, Coef, W1.reshape(E, D, F),
      b1.reshape(E * NFs, 1, BF), W2.reshape(E, F, D), b2.reshape(E, 1, D))


def _sc_combine_body(n_chunks, cc, d, ys_hbm, p0_hbm, p1_hbm,
                     out_hbm, p0_v, p1_v, r0_v, r1_v, sem):
    wid = lax.axis_index("s") * NC + lax.axis_index("c")
    pltpu.sync_copy(p0_hbm.at[wid], p0_v)
    pltpu.sync_copy(p1_hbm.at[wid], p1_v)
    base = wid * (n_chunks * cc)
    nv = d // 16

    def chunk(c, _):
        cp0 = pltpu.async_copy(ys_hbm.at[p0_v.at[c]], r0_v, sem)
        cp1 = pltpu.async_copy(ys_hbm.at[p1_v.at[c]], r1_v, sem)
        cp0.wait()
        cp1.wait()

        def row(r, _):
            for j in range(nv):
                a = r0_v[r, pl.ds(j * 16, 16)]
                b = r1_v[r, pl.ds(j * 16, 16)]
                r0_v[r, pl.ds(j * 16, 16)] = a + b
            return 0

        lax.fori_loop(0, cc, row, 0)
        pltpu.sync_copy(r0_v, out_hbm.at[pl.ds(base + c * cc, cc)])
        return 0

    lax.fori_loop(0, n_chunks, chunk, 0)


def _sc_combine(Ys, pos0, pos1):
    P_cap, D = Ys.shape
    N = pos0.shape[0]
    per_w = N // NW
    cc = 32
    n_chunks = per_w // cc
    p0 = pos0.reshape(NW, n_chunks, cc)
    p1 = pos1.reshape(NW, n_chunks, cc)
    mesh = plsc.VectorSubcoreMesh(core_axis_name="c", subcore_axis_name="s")
    return pl.kernel(
        functools.partial(_sc_combine_body, n_chunks, cc, D),
        out_type=jax.ShapeDtypeStruct((N, D), jnp.float32),
        mesh=mesh,
        scratch_types=[
            pltpu.VMEM((n_chunks, cc), jnp.int32),
            pltpu.VMEM((n_chunks, cc), jnp.int32),
            pltpu.VMEM((cc, D), jnp.float32),
            pltpu.VMEM((cc, D), jnp.float32),
            pltpu.SemaphoreType.DMA,
        ],
    )(Ys, p0, p1)


def kernel(X, Wr, br, W1, b1, W2, b2):
    B, T, D = X.shape
    N = B * T
    P_cap = N * TOP_K + E * BM
    M = P_cap // BM
    Xf = X.reshape(N, D)

    eid0, eid1, rk0, rk1, c0, c1, counts, aux = _router(Xf, Wr, br)

    counts = counts.reshape(E)
    nb = (counts + BM - 1) // BM
    blk_starts = jnp.concatenate(
        [jnp.zeros((1,), jnp.int32), jnp.cumsum(nb)[:-1]]).astype(jnp.int32)
    starts = blk_starts * BM
    pos0 = (starts[eid0.reshape(-1)] + rk0.reshape(-1)).astype(jnp.int32)
    pos1 = (starts[eid1.reshape(-1)] + rk1.reshape(-1)).astype(jnp.int32)
    block_expert = (jnp.searchsorted(
        blk_starts, jnp.arange(M, dtype=jnp.int32), side="right") - 1
    ).astype(jnp.int32)
    n_real = jnp.sum(nb).astype(jnp.int32)
    block_expert_ext = jnp.concatenate([block_expert, n_real[None]])
    Xs, Coef = _sc_scatter(Xf, pos0, pos1, c0, c1, P_cap)
    Ys = _ffn(Xs, Coef, W1, b1, W2, b2, block_expert_ext)
    out = _sc_combine(Ys, pos0, pos1)
    return out.reshape(B, T, D), aux[0, 0]
